# baseline (speedup 1.0000x reference)
import functools

import jax
import jax.numpy as jnp
from jax import lax
from jax.experimental import pallas as pl
from jax.experimental.pallas import tpu as pltpu
from jax.experimental.pallas import tpu_sc as plsc

D = 1024
L = 16
NG = D // L
NC, NS = 2, 16
NW = NC * NS
C = 32
NBUF = 3
UNROLL = 8
EPS = 1e-6


def _rsqrt(x):
    i = plsc.bitcast(x, jnp.int32)
    i = jnp.int32(0x5F3759DF) - lax.shift_right_logical(i, 1)
    y = plsc.bitcast(i, jnp.float32)
    for _ in range(3):
        y = y * (1.5 - 0.5 * x * y * y)
    return y


@functools.lru_cache(maxsize=None)
def _build(B, T, V):
    N = B * T
    per_w = N // NW
    nch = per_w // C

    mesh = plsc.VectorSubcoreMesh(
        core_axis_name="c", subcore_axis_name="s",
        num_cores=NC, num_subcores=NS)

    @functools.partial(
        pl.kernel,
        out_type=jax.ShapeDtypeStruct((N, D), jnp.float32),
        mesh=mesh,
        compiler_params=pltpu.CompilerParams(
            needs_layout_passes=False, use_tc_tiling_on_sc=False),
        scratch_types=[
            pltpu.VMEM((per_w,), jnp.int32),
            [pltpu.VMEM((C, D), jnp.float32) for _ in range(NBUF)],
            pltpu.VMEM((D,), jnp.float32),
            pltpu.VMEM((D,), jnp.float32),
            [pltpu.SemaphoreType.DMA for _ in range(NBUF)],
            [pltpu.SemaphoreType.DMA for _ in range(NBUF)],
            [pltpu.SemaphoreType.DMA for _ in range(NBUF)],
        ],
    )
    def sc_kernel(idx_hbm, wte_hbm, wpe_hbm, scale_hbm, bias_hbm, out_hbm,
                  idx_v, bufs, scale_v, bias_v, wsems, gsems, osems):
        wid = lax.axis_index("s") * NC + lax.axis_index("c")
        base = wid * per_w
        t0 = base % T

        pltpu.sync_copy(idx_hbm.at[pl.ds(base, per_w)], idx_v)
        pltpu.sync_copy(scale_hbm, scale_v)
        pltpu.sync_copy(bias_hbm, bias_v)

        wdesc = [None] * NBUF
        gdesc = [None] * NBUF
        odesc = [None] * NBUF

        def issue_wpe(c):
            b = c % NBUF
            wdesc[b] = pltpu.async_copy(
                wpe_hbm.at[pl.ds(t0 + c * C, C)], bufs[b], wsems[b])

        def issue_gather(c):
            b = c % NBUF
            wdesc[b].wait()
            gdesc[b] = pltpu.async_copy(
                wte_hbm.at[idx_v.at[pl.ds(c * C, C)]], bufs[b], gsems[b],
                add=True)

        def ln_rows(buf):
            zero = jnp.zeros((L,), jnp.float32)
            lanes = lax.iota(jnp.int32, L)

            for r0 in range(0, C, L):
                rows = lanes + r0

                def p1(i, acc):
                    out = []
                    for u in range(UNROLL):
                        j = i * UNROLL + u
                        cols = jnp.full((L,), j, jnp.int32)
                        v = plsc.load_gather(buf, [rows, cols])
                        s, ss = acc[u]
                        out.append((s + v, ss + v * v))
                    return tuple(out)

                acc = lax.fori_loop(
                    0, D // UNROLL, p1,
                    tuple((zero, zero) for _ in range(UNROLL)),
                    unroll=1)
                s = acc[0][0]
                ss = acc[0][1]
                for u in range(1, UNROLL):
                    s = s + acc[u][0]
                    ss = ss + acc[u][1]
                mean = s * (1.0 / D)
                var = ss * (1.0 / D) - mean * mean
                a = _rsqrt(var + EPS)
                m = mean * a

                def p2(i, carry):
                    for u in range(UNROLL):
                        j = i * UNROLL + u
                        cols = jnp.full((L,), j, jnp.int32)
                        v = plsc.load_gather(buf, [rows, cols])
                        sc = plsc.load_gather(scale_v, [cols])
                        bi = plsc.load_gather(bias_v, [cols])
                        y = (v * a - m) * sc + bi
                        plsc.store_scatter(buf, [rows, cols], y)
                    return carry

                lax.fori_loop(0, D // UNROLL, p2, 0, unroll=1)

        issue_wpe(0)
        if nch > 1:
            issue_wpe(1)
        issue_gather(0)
        for c in range(nch):
            b = c % NBUF
            if c + 2 < nch:
                b2 = (c + 2) % NBUF
                if odesc[b2] is not None:
                    odesc[b2].wait()
                    odesc[b2] = None
                issue_wpe(c + 2)
            if c + 1 < nch:
                issue_gather(c + 1)
            gdesc[b].wait()
            ln_rows(bufs[b])
            odesc[b] = pltpu.async_copy(
                bufs[b], out_hbm.at[pl.ds(base + c * C, C)], osems[b])
        for b in range(NBUF):
            if odesc[b] is not None:
                odesc[b].wait()

    return sc_kernel


def kernel(idx, wte, wpe, ln_scale, ln_bias):
    B, T = idx.shape
    k = _build(B, T, wte.shape[0])
    out = k(idx.reshape(-1), wte, wpe, ln_scale, ln_bias)
    return out.reshape(B, T, D)


# speedup vs baseline: 1.7402x; 1.7402x over previous
import functools

import jax
import jax.numpy as jnp
from jax import lax
from jax.experimental import pallas as pl
from jax.experimental.pallas import tpu as pltpu
from jax.experimental.pallas import tpu_sc as plsc

D = 1024
L = 16
NG = D // L
NC, NS = 2, 16
NW = NC * NS
C = 32
NBUF = 3
UNROLL = 8
EPS = 1e-6


def _rsqrt(x):
    cast = lax.bitcast_convert_type if jnp.ndim(x) == 0 else plsc.bitcast
    i = cast(x, jnp.int32)
    i = jnp.int32(0x5F3759DF) - lax.shift_right_logical(i, 1)
    y = cast(i, jnp.float32)
    for _ in range(3):
        y = y * (1.5 - 0.5 * x * y * y)
    return y


@functools.lru_cache(maxsize=None)
def _build(B, T, V):
    N = B * T
    per_w = N // NW
    nch = per_w // C

    mesh = plsc.VectorSubcoreMesh(
        core_axis_name="c", subcore_axis_name="s",
        num_cores=NC, num_subcores=NS)

    @functools.partial(
        pl.kernel,
        out_type=jax.ShapeDtypeStruct((N, D), jnp.float32),
        mesh=mesh,
        compiler_params=pltpu.CompilerParams(
            needs_layout_passes=False, use_tc_tiling_on_sc=False),
        scratch_types=[
            pltpu.VMEM((per_w,), jnp.int32),
            [pltpu.VMEM((C, D), jnp.float32) for _ in range(NBUF)],
            pltpu.VMEM((D,), jnp.float32),
            pltpu.VMEM((D,), jnp.float32),
            [pltpu.SemaphoreType.DMA for _ in range(NBUF)],
            [pltpu.SemaphoreType.DMA for _ in range(NBUF)],
            [pltpu.SemaphoreType.DMA for _ in range(NBUF)],
        ],
    )
    def sc_kernel(idx_hbm, wte_hbm, wpe_hbm, scale_hbm, bias_hbm, out_hbm,
                  idx_v, bufs, scale_v, bias_v, wsems, gsems, osems):
        wid = lax.axis_index("s") * NC + lax.axis_index("c")
        base = wid * per_w
        t0 = base % T

        pltpu.sync_copy(idx_hbm.at[pl.ds(base, per_w)], idx_v)
        pltpu.sync_copy(scale_hbm, scale_v)
        pltpu.sync_copy(bias_hbm, bias_v)

        wdesc = [None] * NBUF
        gdesc = [None] * NBUF
        odesc = [None] * NBUF

        def issue_wpe(c):
            b = c % NBUF
            wdesc[b] = pltpu.async_copy(
                wpe_hbm.at[pl.ds(t0 + c * C, C)], bufs[b], wsems[b])

        def issue_gather(c):
            b = c % NBUF
            wdesc[b].wait()
            gdesc[b] = pltpu.async_copy(
                wte_hbm.at[idx_v.at[pl.ds(c * C, C)]], bufs[b], gsems[b],
                add=True)

        def ln_rows(buf):
            zero = jnp.zeros((L,), jnp.float32)

            def row_fn(r, _):
                def p1(i, acc):
                    out = []
                    for u in range(UNROLL):
                        v = buf[r, pl.ds((i * UNROLL + u) * L, L)]
                        s, ss = acc[u]
                        out.append((s + v, ss + v * v))
                    return tuple(out)

                acc = lax.fori_loop(
                    0, NG // UNROLL, p1,
                    tuple((zero, zero) for _ in range(UNROLL)))
                s = acc[0][0]
                ss = acc[0][1]
                for u in range(1, UNROLL):
                    s = s + acc[u][0]
                    ss = ss + acc[u][1]
                tot = jnp.sum(s)
                tot2 = jnp.sum(ss)
                mean = tot * (1.0 / D)
                var = tot2 * (1.0 / D) - mean * mean
                a = _rsqrt(var + EPS)
                m = mean * a

                def p2(i, carry):
                    for u in range(UNROLL):
                        col = (i * UNROLL + u) * L
                        v = buf[r, pl.ds(col, L)]
                        y = v * a - m
                        y = y * scale_v[pl.ds(col, L)] + bias_v[pl.ds(col, L)]
                        buf[r, pl.ds(col, L)] = y
                    return carry

                lax.fori_loop(0, NG // UNROLL, p2, 0)
                return 0

            lax.fori_loop(0, C, row_fn, 0)

        issue_wpe(0)
        if nch > 1:
            issue_wpe(1)
        issue_gather(0)
        for c in range(nch):
            b = c % NBUF
            if c + 2 < nch:
                b2 = (c + 2) % NBUF
                if odesc[b2] is not None:
                    odesc[b2].wait()
                    odesc[b2] = None
                issue_wpe(c + 2)
            if c + 1 < nch:
                issue_gather(c + 1)
            gdesc[b].wait()
            ln_rows(bufs[b])
            odesc[b] = pltpu.async_copy(
                bufs[b], out_hbm.at[pl.ds(base + c * C, C)], osems[b])
        for b in range(NBUF):
            if odesc[b] is not None:
                odesc[b].wait()

    return sc_kernel


def kernel(idx, wte, wpe, ln_scale, ln_bias):
    B, T = idx.shape
    k = _build(B, T, wte.shape[0])
    out = k(idx.reshape(-1), wte, wpe, ln_scale, ln_bias)
    return out.reshape(B, T, D)


# speedup vs baseline: 4.1926x; 2.4093x over previous
import functools

import jax
import jax.numpy as jnp
from jax import lax
from jax.experimental import pallas as pl
from jax.experimental.pallas import tpu as pltpu
from jax.experimental.pallas import tpu_sc as plsc

D = 1024
L = 16
NG = D // L
NC, NS = 2, 16
NW = NC * NS
C = 16
NBUF = 3
UNROLL = 8
EPS = 1e-6


def _rsqrt(x):
    cast = lax.bitcast_convert_type if jnp.ndim(x) == 0 else plsc.bitcast
    i = cast(x, jnp.int32)
    i = jnp.int32(0x5F3759DF) - lax.shift_right_logical(i, 1)
    y = cast(i, jnp.float32)
    for _ in range(3):
        y = y * (1.5 - 0.5 * x * y * y)
    return y


@functools.lru_cache(maxsize=None)
def _build(B, T, V):
    N = B * T
    per_w = N // NW
    nch = per_w // C
    per_b = T // per_w * 0 + per_w
    wpb = T // per_w

    mesh = plsc.VectorSubcoreMesh(
        core_axis_name="c", subcore_axis_name="s",
        num_cores=NC, num_subcores=NS)

    @functools.partial(
        pl.kernel,
        out_type=jax.ShapeDtypeStruct((N, D), jnp.float32),
        mesh=mesh,
        compiler_params=pltpu.CompilerParams(
            needs_layout_passes=False, use_tc_tiling_on_sc=True),
        scratch_types=[
            pltpu.VMEM((per_w,), jnp.int32),
            [pltpu.VMEM((C, D), jnp.float32) for _ in range(NBUF)],
            [pltpu.VMEM((C, D), jnp.float32) for _ in range(NBUF)],
            pltpu.VMEM((D,), jnp.float32),
            pltpu.VMEM((D,), jnp.float32),
            [pltpu.SemaphoreType.DMA for _ in range(NBUF)],
            [pltpu.SemaphoreType.DMA for _ in range(NBUF)],
            [pltpu.SemaphoreType.DMA for _ in range(NBUF)],
        ],
    )
    def sc_kernel(idx_hbm, wte_hbm, wpe_hbm, scale_hbm, bias_hbm, out_hbm,
                  idx_v, bufs, wbufs, scale_v, bias_v, wsems, gsems, osems):
        wid = lax.axis_index("s") * NC + lax.axis_index("c")
        base = wid * per_w
        brow = wid // wpb
        t0 = (wid % wpb) * per_w

        pltpu.sync_copy(idx_hbm.at[brow, pl.ds(t0, per_w)], idx_v)
        pltpu.sync_copy(scale_hbm, scale_v)
        pltpu.sync_copy(bias_hbm, bias_v)

        lanes = lax.iota(jnp.int32, L)

        wdesc = [None] * NBUF
        gdesc = [None] * NBUF
        odesc = [None] * NBUF

        def issue_chunk(c):
            b = c % NBUF
            idx_vec = idx_v[pl.ds(c * C, C)]
            gdesc[b] = pltpu.async_copy(
                wte_hbm.at[idx_vec], bufs[b], gsems[b])
            wdesc[b] = pltpu.async_copy(
                wpe_hbm.at[pl.ds(t0 + c * C, C)], wbufs[b], wsems[b])

        def ln_rows(buf, wbuf):
            zero = jnp.zeros((L,), jnp.float32)

            def row_fn(r, _):
                def p1(i, acc):
                    out = []
                    for u in range(UNROLL):
                        sl = pl.ds((i * UNROLL + u) * L, L)
                        v = buf[r, sl] + wbuf[r, sl]
                        buf[r, sl] = v
                        s, ss = acc[u]
                        out.append((s + v, ss + v * v))
                    return tuple(out)

                acc = lax.fori_loop(
                    0, NG // UNROLL, p1,
                    tuple((zero, zero) for _ in range(UNROLL)))
                s = acc[0][0]
                ss = acc[0][1]
                for u in range(1, UNROLL):
                    s = s + acc[u][0]
                    ss = ss + acc[u][1]
                tot = jnp.sum(s)
                tot2 = jnp.sum(ss)
                mean = tot * (1.0 / D)
                var = tot2 * (1.0 / D) - mean * mean
                a = _rsqrt(var + EPS)
                m = mean * a

                @plsc.parallel_loop(0, NG, step=UNROLL)
                def p2(i):
                    for u in range(UNROLL):
                        sl = pl.ds((i + u) * L, L)
                        y = buf[r, sl] * a - m
                        buf[r, sl] = (y * scale_v[pl.ds((i + u) * L, L)]
                                      + bias_v[pl.ds((i + u) * L, L)])

                return 0

            lax.fori_loop(0, C, row_fn, 0)

        for c in range(min(2, nch)):
            issue_chunk(c)
        for c in range(nch):
            b = c % NBUF
            if c + 2 < nch:
                b2 = (c + 2) % NBUF
                if odesc[b2] is not None:
                    odesc[b2].wait()
                    odesc[b2] = None
                issue_chunk(c + 2)
            gdesc[b].wait()
            wdesc[b].wait()
            ln_rows(bufs[b], wbufs[b])
            row_vec = base + c * C + lanes
            odesc[b] = pltpu.async_copy(
                bufs[b], out_hbm.at[row_vec], osems[b])
        for b in range(NBUF):
            if odesc[b] is not None:
                odesc[b].wait()

    return sc_kernel


def kernel(idx, wte, wpe, ln_scale, ln_bias):
    B, T = idx.shape
    k = _build(B, T, wte.shape[0])
    out = k(idx, wte, wpe, ln_scale, ln_bias)
    return out.reshape(B, T, D)
